# baseline (speedup 1.0000x reference)
import jax
import jax.numpy as jnp
from jax import lax
from jax.experimental import pallas as pl
from jax.experimental.pallas import tpu as pltpu
from jax.experimental.pallas import tpu_sc as plsc

_M = 128
_N = 32768
_K = int(0.5 * (_N - 1))
_NC = 2
_NS = 16
_NW = _NC * _NS
_RS = 32
_RPW = _RS // _NW
_TB = 32
_L = 16
_CHUNKS = _N // _L
_U = 8
_INT_MIN = -2147483648
_NEG_INF = float("-inf")


def _splat(v, dtype=jnp.int32):
    return jnp.full((_L,), v, dtype)


def _tree_reduce(red_v, v, neutral, op, dtype=jnp.float32):
    red_v[pl.ds(_L, _L)] = _splat(neutral, dtype)
    red_v[pl.ds(0, _L)] = v
    a = op(v, red_v[pl.ds(8, _L)])
    red_v[pl.ds(0, _L)] = a
    a = op(a, red_v[pl.ds(4, _L)])
    red_v[pl.ds(0, _L)] = a
    a = op(a, red_v[pl.ds(2, _L)])
    red_v[pl.ds(0, _L)] = a
    a = op(a, red_v[pl.ds(1, _L)])
    return a[0]


def _to_key(ib):
    return ib ^ ((ib >> 31) & 0x7FFFFFFF)


def _sc_body(inputs_hbm, targets_hbm, out_hbm, row_v, keys_v, tgt_v, out_v,
             red_v, redi_v):
    wid = lax.axis_index("s") * _NC + lax.axis_index("c")
    pltpu.sync_copy(targets_hbm, tgt_v)
    iota = lax.iota(jnp.int32, _L)
    zeros_f = jnp.zeros((_L,), jnp.float32)
    ones_f = jnp.full((_L,), 1.0, jnp.float32)
    acc = zeros_f
    fmax = lambda a, b: jnp.maximum(a, b)
    fadd = lambda a, b: a + b

    tgt_base = pl.multiple_of((wid * _RPW // _L) * _L, _L)
    tgt_blk = tgt_v[pl.ds(tgt_base, _L)].astype(jnp.float32)

    for j in range(_RPW):
        r = wid * _RPW + j
        pltpu.sync_copy(inputs_hbm.at[r], row_v)
        tgt_s = _tree_reduce(
            red_v,
            jnp.where(iota == _splat(r % _L), tgt_blk, _splat(-1.0, jnp.float32)),
            _NEG_INF, fmax).astype(jnp.int32)

        pos_base = pl.multiple_of((tgt_s // _L) * _L, _L)
        pos_blk = row_v[pl.ds(pos_base, _L)]
        pos_s = _tree_reduce(
            red_v,
            jnp.where(iota == _splat(tgt_s % _L), pos_blk,
                      _splat(_NEG_INF, jnp.float32)),
            _NEG_INF, fmax)
        pos_v = _splat(pos_s, jnp.float32)
        pos_key = _tree_reduce(
            redi_v,
            jnp.where(iota == _splat(tgt_s % _L),
                      _to_key(lax.bitcast_convert_type(pos_blk, jnp.int32)),
                      _splat(_INT_MIN)),
            _INT_MIN, fmax, jnp.int32)

        def build_body(c, maxacc):
            for u in range(_U):
                off = c * (_U * _L) + u * _L
                x = row_v[pl.ds(off, _L)]
                keys_v[pl.ds(off, _L)] = _to_key(
                    lax.bitcast_convert_type(x, jnp.int32))
                maxacc = jnp.maximum(maxacc, x)
            return maxacc

        maxacc = lax.fori_loop(0, _CHUNKS // _U, build_body,
                               _splat(_NEG_INF, jnp.float32))
        mx_s = _tree_reduce(red_v, maxacc, _NEG_INF, fmax)
        mx_v = _splat(mx_s, jnp.float32)

        def search_body(i, p):
            b = 31 - i
            cand = jnp.where(i == 0, 0, p | (jnp.int32(1) << b))
            cand_v = _splat(cand)

            @plsc.parallel_loop(0, _CHUNKS, unroll=_U, carry=zeros_f)
            def cnt_loop(c, cnt):
                kc = keys_v[pl.ds(c * _L, _L)]
                return cnt + jnp.where(kc >= cand_v, ones_f, zeros_f)

            cnt = _tree_reduce(red_v, cnt_loop, 0.0, fadd)
            cnt = cnt - jnp.where(pos_key >= cand, 1.0, 0.0)
            return jnp.where(cnt >= float(_K), cand, p)

        kth = lax.fori_loop(0, 32, search_body, jnp.int32(_INT_MIN))
        kth_v = _splat(kth)

        def sum_body(c, carry):
            s, cg = carry
            for u in range(_U):
                off = c * (_U * _L) + u * _L
                kc = keys_v[pl.ds(off, _L)]
                xc = row_v[pl.ds(off, _L)]
                gt = kc > kth_v
                e = jnp.exp((xc - mx_v) * 10.0)
                s = s + jnp.where(gt, e, zeros_f)
                cg = cg + jnp.where(gt, ones_f, zeros_f)
            return (s, cg)

        s, cg = lax.fori_loop(0, _CHUNKS // _U, sum_body, (zeros_f, zeros_f))
        s_all = _tree_reduce(red_v, s, 0.0, fadd)
        cg_all = _tree_reduce(red_v, cg, 0.0, fadd)

        e_pos_v = jnp.exp((pos_v - mx_v) * 10.0)
        thr_v = lax.bitcast_convert_type(_to_key(kth_v), jnp.float32)
        e_thr_v = jnp.exp((thr_v - mx_v) * 10.0)
        pos_gt_v = _splat(jnp.where(pos_key > kth, 1.0, 0.0), jnp.float32)
        total_v = (_splat(s_all, jnp.float32) - pos_gt_v * e_pos_v
                   + (float(_K) - (_splat(cg_all, jnp.float32) - pos_gt_v))
                   * e_thr_v + e_pos_v)

        acc = jnp.where(iota == _splat(j), total_v, acc)
        acc = jnp.where(iota == _splat(_RPW + j), mx_v, acc)
        acc = jnp.where(iota == _splat(2 * _RPW + j), pos_v, acc)

    out_v[...] = acc
    pltpu.sync_copy(out_v, out_hbm.at[wid])


def _sc_stage(inputs, targets):
    mesh = plsc.VectorSubcoreMesh(core_axis_name="c", subcore_axis_name="s",
                                  num_cores=_NC)
    return pl.kernel(
        _sc_body,
        out_type=jax.ShapeDtypeStruct((_NW, _L), jnp.float32),
        mesh=mesh,
        scratch_types=[
            pltpu.VMEM((_N,), jnp.float32),
            pltpu.VMEM((_N,), jnp.int32),
            pltpu.VMEM((_M,), jnp.int32),
            pltpu.VMEM((_L,), jnp.float32),
            pltpu.VMEM((2 * _L,), jnp.float32),
            pltpu.VMEM((2 * _L,), jnp.int32),
        ],
    )(inputs, targets)


def _tc_body(x_ref, t_ref, o_ref, keys_ref):
    x = x_ref[...]
    tgt = t_ref[...]
    col = lax.broadcasted_iota(jnp.int32, (_TB, _N), 1)
    ispos = col == tgt
    keys_ref[...] = jnp.where(
        ispos, _INT_MIN, _to_key(lax.bitcast_convert_type(x, jnp.int32)))
    mx = jnp.max(x, axis=1, keepdims=True)
    pos = jnp.sum(jnp.where(ispos, x, 0.0), axis=1, keepdims=True)

    def search_body(i, p):
        cand = jnp.where(i == 0, jnp.zeros_like(p),
                         p | (jnp.int32(1) << (31 - i)))
        cnt = jnp.sum((keys_ref[...] >= cand).astype(jnp.float32),
                      axis=1, keepdims=True)
        return jnp.where(cnt >= float(_K), cand, p)

    kth = lax.fori_loop(
        0, 32, search_body, jnp.full((_TB, 1), _INT_MIN, jnp.int32))

    keys = keys_ref[...]
    gt = keys > kth
    e = jnp.exp((x - mx) * 10.0)
    s_gt = jnp.sum(jnp.where(gt, e, 0.0), axis=1, keepdims=True)
    cnt_gt = jnp.sum(gt.astype(jnp.float32), axis=1, keepdims=True)
    thr = lax.bitcast_convert_type(_to_key(kth), jnp.float32)
    e_thr = jnp.exp((thr - mx) * 10.0)
    e_pos = jnp.exp((pos - mx) * 10.0)
    total = s_gt + (float(_K) - cnt_gt) * e_thr + e_pos
    o_ref[...] = jnp.log(total) + 10.0 * (mx - pos)


def _tc_stage(inputs, targets_2d):
    nblk = (_M - _RS) // _TB
    return pl.pallas_call(
        _tc_body,
        grid=(nblk,),
        in_specs=[
            pl.BlockSpec((_TB, _N), lambda i: (i + _RS // _TB, 0)),
            pl.BlockSpec((_TB, 1), lambda i: (i + _RS // _TB, 0)),
        ],
        out_specs=pl.BlockSpec((_TB, 1), lambda i: (i, 0)),
        out_shape=jax.ShapeDtypeStruct((_M - _RS, 1), jnp.float32),
        scratch_shapes=[pltpu.VMEM((_TB, _N), jnp.int32)],
    )(inputs, targets_2d)


def _finish_body(sc_ref, tc_ref, o_ref):
    sc = sc_ref[...]
    s = sc[:, 0:_RPW]
    mx = sc[:, _RPW:2 * _RPW]
    pos = sc[:, 2 * _RPW:3 * _RPW]
    sc_loss = jnp.log(s) + 10.0 * (mx - pos)
    o_ref[0] = (jnp.sum(sc_loss) + jnp.sum(tc_ref[...])) * (1.0 / _M)


def _finish(sc_out, tc_out):
    return pl.pallas_call(
        _finish_body,
        out_shape=jax.ShapeDtypeStruct((1,), jnp.float32),
        out_specs=pl.BlockSpec(memory_space=pltpu.SMEM),
    )(sc_out, tc_out)


@jax.jit
def _run(inputs, targets):
    t32 = targets.astype(jnp.int32)
    sc_out = _sc_stage(inputs, t32)
    tc_out = _tc_stage(inputs, t32.reshape(_M, 1))
    return _finish(sc_out, tc_out)[0]


def kernel(inputs, targets):
    return _run(inputs, targets)


# speedup vs baseline: 2.4378x; 2.4378x over previous
import jax
import jax.numpy as jnp
from jax import lax
from jax.experimental import pallas as pl
from jax.experimental.pallas import tpu as pltpu
from jax.experimental.pallas import tpu_sc as plsc

_M = 128
_N = 32768
_K = int(0.5 * (_N - 1))
_NC = 2
_NS = 16
_NW = _NC * _NS
_RS = 32
_RPW = _RS // _NW
_TB = 32
_L = 16
_CHUNKS = _N // _L
_U = 8
_INT_MIN = -2147483648
_NEG_INF = float("-inf")


def _splat(v, dtype=jnp.int32):
    return jnp.full((_L,), v, dtype)


def _tree_reduce(red_v, v, neutral, op, dtype=jnp.float32):
    red_v[pl.ds(_L, _L)] = _splat(neutral, dtype)
    red_v[pl.ds(0, _L)] = v
    a = op(v, red_v[pl.ds(8, _L)])
    red_v[pl.ds(0, _L)] = a
    a = op(a, red_v[pl.ds(4, _L)])
    red_v[pl.ds(0, _L)] = a
    a = op(a, red_v[pl.ds(2, _L)])
    red_v[pl.ds(0, _L)] = a
    a = op(a, red_v[pl.ds(1, _L)])
    return a[0]


def _to_key(ib):
    return ib ^ ((ib >> 31) & 0x7FFFFFFF)


def _sc_body(inputs_hbm, targets_hbm, out_hbm, row_v, keys_v, tgt_v, out_v,
             red_v, redi_v):
    wid = lax.axis_index("s") * _NC + lax.axis_index("c")
    pltpu.sync_copy(targets_hbm, tgt_v)
    iota = lax.iota(jnp.int32, _L)
    zeros_f = jnp.zeros((_L,), jnp.float32)
    ones_f = jnp.full((_L,), 1.0, jnp.float32)
    acc = zeros_f
    fmax = lambda a, b: jnp.maximum(a, b)
    fadd = lambda a, b: a + b

    tgt_base = pl.multiple_of((wid * _RPW // _L) * _L, _L)
    tgt_blk = tgt_v[pl.ds(tgt_base, _L)].astype(jnp.float32)

    for j in range(_RPW):
        r = wid * _RPW + j
        pltpu.sync_copy(inputs_hbm.at[r], row_v)
        tgt_s = _tree_reduce(
            red_v,
            jnp.where(iota == _splat(r % _L), tgt_blk, _splat(-1.0, jnp.float32)),
            _NEG_INF, fmax).astype(jnp.int32)

        pos_base = pl.multiple_of((tgt_s // _L) * _L, _L)
        pos_blk = row_v[pl.ds(pos_base, _L)]
        pos_s = _tree_reduce(
            red_v,
            jnp.where(iota == _splat(tgt_s % _L), pos_blk,
                      _splat(_NEG_INF, jnp.float32)),
            _NEG_INF, fmax)
        pos_v = _splat(pos_s, jnp.float32)
        pos_key = _tree_reduce(
            redi_v,
            jnp.where(iota == _splat(tgt_s % _L),
                      _to_key(lax.bitcast_convert_type(pos_blk, jnp.int32)),
                      _splat(_INT_MIN)),
            _INT_MIN, fmax, jnp.int32)

        def build_body(c, carry):
            maxacc, cnt0 = carry
            for u in range(_U):
                off = c * (_U * _L) + u * _L
                x = row_v[pl.ds(off, _L)]
                key = _to_key(lax.bitcast_convert_type(x, jnp.int32))
                keys_v[pl.ds(off, _L)] = key
                maxacc = jnp.maximum(maxacc, x)
                cnt0 = cnt0 + jnp.where(key >= _splat(0), ones_f, zeros_f)
            return maxacc, cnt0

        maxacc, cnt0 = lax.fori_loop(
            0, _CHUNKS // _U, build_body,
            (_splat(_NEG_INF, jnp.float32), zeros_f))
        mx_s = _tree_reduce(red_v, maxacc, _NEG_INF, fmax)
        mx_v = _splat(mx_s, jnp.float32)
        cnt0_s = (_tree_reduce(red_v, cnt0, 0.0, fadd)
                  - jnp.where(pos_key >= 0, 1.0, 0.0))
        acc0 = cnt0_s >= float(_K)
        p_init = jnp.where(acc0, jnp.int32(0), jnp.int32(_INT_MIN))
        c_init = jnp.where(acc0, cnt0_s, float(_N - 1))

        def search_body(i, carry):
            p, cntp = carry
            cand = p | (jnp.int32(1) << (31 - i))
            cand_v = _splat(cand)

            def cnt_body(c, cnt):
                for u in range(_U):
                    kc = keys_v[pl.ds(c * (_U * _L) + u * _L, _L)]
                    cnt = cnt + jnp.where(kc >= cand_v, ones_f, zeros_f)
                return cnt

            cnt = _tree_reduce(
                red_v, lax.fori_loop(0, _CHUNKS // _U, cnt_body, zeros_f),
                0.0, fadd)
            cnt = cnt - jnp.where(pos_key >= cand, 1.0, 0.0)
            ok = cnt >= float(_K)
            return (jnp.where(ok, cand, p), jnp.where(ok, cnt, cntp))

        kth, cnt_ge = lax.fori_loop(1, 32, search_body, (p_init, c_init))
        kth_v = _splat(kth)

        thr_v = lax.bitcast_convert_type(_to_key(kth_v), jnp.float32)

        def sum_body(c, s):
            for u in range(_U):
                xc = row_v[pl.ds(c * (_U * _L) + u * _L, _L)]
                e = jnp.exp((xc - mx_v) * 10.0)
                s = s + jnp.where(xc >= thr_v, e, zeros_f)
            return s

        s = lax.fori_loop(0, _CHUNKS // _U, sum_body, zeros_f)
        s_all = _tree_reduce(red_v, s, 0.0, fadd)

        e_pos_v = jnp.exp((pos_v - mx_v) * 10.0)
        e_thr_v = jnp.exp((thr_v - mx_v) * 10.0)
        pos_ge_v = _splat(jnp.where(pos_key >= kth, 1.0, 0.0), jnp.float32)
        total_v = (_splat(s_all, jnp.float32) - pos_ge_v * e_pos_v
                   + (float(_K) - _splat(cnt_ge, jnp.float32))
                   * e_thr_v + e_pos_v)

        acc = jnp.where(iota == _splat(j), total_v, acc)
        acc = jnp.where(iota == _splat(_RPW + j), mx_v, acc)
        acc = jnp.where(iota == _splat(2 * _RPW + j), pos_v, acc)

    out_v[...] = acc
    pltpu.sync_copy(out_v, out_hbm.at[wid])


def _sc_stage(inputs, targets):
    mesh = plsc.VectorSubcoreMesh(core_axis_name="c", subcore_axis_name="s",
                                  num_cores=_NC)
    return pl.kernel(
        _sc_body,
        out_type=jax.ShapeDtypeStruct((_NW, _L), jnp.float32),
        mesh=mesh,
        scratch_types=[
            pltpu.VMEM((_N,), jnp.float32),
            pltpu.VMEM((_N,), jnp.int32),
            pltpu.VMEM((_M,), jnp.int32),
            pltpu.VMEM((_L,), jnp.float32),
            pltpu.VMEM((2 * _L,), jnp.float32),
            pltpu.VMEM((2 * _L,), jnp.int32),
        ],
    )(inputs, targets)


def _tc_body(x_ref, t_ref, o_ref, keys_ref):
    x = x_ref[...]
    tgt = t_ref[...]
    col = lax.broadcasted_iota(jnp.int32, (_TB, _N), 1)
    ispos = col == tgt
    keys_ref[...] = jnp.where(
        ispos, _INT_MIN, _to_key(lax.bitcast_convert_type(x, jnp.int32)))
    mx = jnp.max(x, axis=1, keepdims=True)
    pos = jnp.sum(jnp.where(ispos, x, 0.0), axis=1, keepdims=True)

    def search_body(i, p):
        cand = jnp.where(i == 0, jnp.zeros_like(p),
                         p | (jnp.int32(1) << (31 - i)))
        cnt = jnp.sum((keys_ref[...] >= cand).astype(jnp.float32),
                      axis=1, keepdims=True)
        return jnp.where(cnt >= float(_K), cand, p)

    kth = lax.fori_loop(
        0, 32, search_body, jnp.full((_TB, 1), _INT_MIN, jnp.int32))

    keys = keys_ref[...]
    gt = keys > kth
    e = jnp.exp((x - mx) * 10.0)
    s_gt = jnp.sum(jnp.where(gt, e, 0.0), axis=1, keepdims=True)
    cnt_gt = jnp.sum(gt.astype(jnp.float32), axis=1, keepdims=True)
    thr = lax.bitcast_convert_type(_to_key(kth), jnp.float32)
    e_thr = jnp.exp((thr - mx) * 10.0)
    e_pos = jnp.exp((pos - mx) * 10.0)
    total = s_gt + (float(_K) - cnt_gt) * e_thr + e_pos
    o_ref[...] = jnp.log(total) + 10.0 * (mx - pos)


def _tc_stage(inputs, targets_2d):
    nblk = (_M - _RS) // _TB
    return pl.pallas_call(
        _tc_body,
        grid=(nblk,),
        in_specs=[
            pl.BlockSpec((_TB, _N), lambda i: (i + _RS // _TB, 0)),
            pl.BlockSpec((_TB, 1), lambda i: (i + _RS // _TB, 0)),
        ],
        out_specs=pl.BlockSpec((_TB, 1), lambda i: (i, 0)),
        out_shape=jax.ShapeDtypeStruct((_M - _RS, 1), jnp.float32),
        scratch_shapes=[pltpu.VMEM((_TB, _N), jnp.int32)],
    )(inputs, targets_2d)


def _finish_body(sc_ref, tc_ref, o_ref):
    sc = sc_ref[...]
    s = sc[:, 0:_RPW]
    mx = sc[:, _RPW:2 * _RPW]
    pos = sc[:, 2 * _RPW:3 * _RPW]
    sc_loss = jnp.log(s) + 10.0 * (mx - pos)
    o_ref[0] = (jnp.sum(sc_loss) + jnp.sum(tc_ref[...])) * (1.0 / _M)


def _finish(sc_out, tc_out):
    return pl.pallas_call(
        _finish_body,
        out_shape=jax.ShapeDtypeStruct((1,), jnp.float32),
        out_specs=pl.BlockSpec(memory_space=pltpu.SMEM),
    )(sc_out, tc_out)


@jax.jit
def _run(inputs, targets):
    t32 = targets.astype(jnp.int32)
    sc_out = _sc_stage(inputs, t32)
    tc_out = _tc_stage(inputs, t32.reshape(_M, 1))
    return _finish(sc_out, tc_out)[0]


def kernel(inputs, targets):
    return _run(inputs, targets)


# speedup vs baseline: 2.6985x; 1.1069x over previous
import jax
import jax.numpy as jnp
from jax import lax
from jax.experimental import pallas as pl
from jax.experimental.pallas import tpu as pltpu
from jax.experimental.pallas import tpu_sc as plsc

_M = 128
_N = 32768
_K = int(0.5 * (_N - 1))
_NC = 2
_NS = 16
_NW = _NC * _NS
_RS = 32
_RPW = _RS // _NW
_TB = 32
_L = 16
_CHUNKS = _N // _L
_U = 8
_INT_MIN = -2147483648
_NEG_INF = float("-inf")


def _splat(v, dtype=jnp.int32):
    return jnp.full((_L,), v, dtype)


def _tree_reduce(red_v, v, neutral, op, dtype=jnp.float32):
    red_v[pl.ds(_L, _L)] = _splat(neutral, dtype)
    red_v[pl.ds(0, _L)] = v
    a = op(v, red_v[pl.ds(8, _L)])
    red_v[pl.ds(0, _L)] = a
    a = op(a, red_v[pl.ds(4, _L)])
    red_v[pl.ds(0, _L)] = a
    a = op(a, red_v[pl.ds(2, _L)])
    red_v[pl.ds(0, _L)] = a
    a = op(a, red_v[pl.ds(1, _L)])
    return a[0]


def _to_key(ib):
    return ib ^ ((ib >> 31) & 0x7FFFFFFF)


def _sc_body(inputs_hbm, targets_hbm, out_hbm, row_v, keys_v, tgt_v, out_v,
             red_v, redi_v):
    wid = lax.axis_index("s") * _NC + lax.axis_index("c")
    pltpu.sync_copy(targets_hbm, tgt_v)
    iota = lax.iota(jnp.int32, _L)
    zeros_f = jnp.zeros((_L,), jnp.float32)
    ones_f = jnp.full((_L,), 1.0, jnp.float32)
    acc = zeros_f
    fmax = lambda a, b: jnp.maximum(a, b)
    fadd = lambda a, b: a + b

    tgt_base = pl.multiple_of((wid * _RPW // _L) * _L, _L)
    tgt_blk = tgt_v[pl.ds(tgt_base, _L)].astype(jnp.float32)

    for j in range(_RPW):
        r = wid * _RPW + j
        pltpu.sync_copy(inputs_hbm.at[r], row_v)
        tgt_s = _tree_reduce(
            red_v,
            jnp.where(iota == _splat(r % _L), tgt_blk, _splat(-1.0, jnp.float32)),
            _NEG_INF, fmax).astype(jnp.int32)

        pos_base = pl.multiple_of((tgt_s // _L) * _L, _L)
        pos_blk = row_v[pl.ds(pos_base, _L)]
        pos_s = _tree_reduce(
            red_v,
            jnp.where(iota == _splat(tgt_s % _L), pos_blk,
                      _splat(_NEG_INF, jnp.float32)),
            _NEG_INF, fmax)
        pos_v = _splat(pos_s, jnp.float32)
        pos_key = _tree_reduce(
            redi_v,
            jnp.where(iota == _splat(tgt_s % _L),
                      _to_key(lax.bitcast_convert_type(pos_blk, jnp.int32)),
                      _splat(_INT_MIN)),
            _INT_MIN, fmax, jnp.int32)

        def build_body(c, carry):
            maxacc, cnt0 = carry
            for u in range(_U):
                off = c * (_U * _L) + u * _L
                x = row_v[pl.ds(off, _L)]
                key = _to_key(lax.bitcast_convert_type(x, jnp.int32))
                keys_v[pl.ds(off, _L)] = key
                maxacc = jnp.maximum(maxacc, x)
                cnt0 = cnt0 + jnp.where(key >= _splat(0), ones_f, zeros_f)
            return maxacc, cnt0

        maxacc, cnt0 = lax.fori_loop(
            0, _CHUNKS // _U, build_body,
            (_splat(_NEG_INF, jnp.float32), zeros_f))
        mx_s = _tree_reduce(red_v, maxacc, _NEG_INF, fmax)
        mx_v = _splat(mx_s, jnp.float32)
        cnt0_s = (_tree_reduce(red_v, cnt0, 0.0, fadd)
                  - jnp.where(pos_key >= 0, 1.0, 0.0))
        acc0 = cnt0_s >= float(_K)
        p_init = jnp.where(acc0, jnp.int32(0), jnp.int32(_INT_MIN))
        c_init = jnp.where(acc0, cnt0_s, float(_N - 1))

        def search_body(i, carry):
            p, cntp = carry
            cand = p | (jnp.int32(1) << (31 - i))
            cand_v = _splat(cand)

            def cnt_body(c, carry):
                ca, cb = carry
                for u in range(0, _U, 2):
                    base = c * (_U * _L) + u * _L
                    ka = keys_v[pl.ds(base, _L)]
                    kb = keys_v[pl.ds(base + _L, _L)]
                    ca = ca + jnp.where(ka >= cand_v, ones_f, zeros_f)
                    cb = cb + jnp.where(kb >= cand_v, ones_f, zeros_f)
                return ca, cb

            ca, cb = lax.fori_loop(0, _CHUNKS // _U, cnt_body,
                                   (zeros_f, zeros_f))
            cnt = _tree_reduce(red_v, ca + cb, 0.0, fadd)
            cnt = cnt - jnp.where(pos_key >= cand, 1.0, 0.0)
            ok = cnt >= float(_K)
            return (jnp.where(ok, cand, p), jnp.where(ok, cnt, cntp))

        kth, cnt_ge = lax.fori_loop(1, 32, search_body, (p_init, c_init))
        kth_v = _splat(kth)

        thr_v = lax.bitcast_convert_type(_to_key(kth_v), jnp.float32)

        def sum_body(c, s):
            for u in range(_U):
                xc = row_v[pl.ds(c * (_U * _L) + u * _L, _L)]
                e = jnp.exp((xc - mx_v) * 10.0)
                s = s + jnp.where(xc >= thr_v, e, zeros_f)
            return s

        s = lax.fori_loop(0, _CHUNKS // _U, sum_body, zeros_f)
        s_all = _tree_reduce(red_v, s, 0.0, fadd)

        e_pos_v = jnp.exp((pos_v - mx_v) * 10.0)
        e_thr_v = jnp.exp((thr_v - mx_v) * 10.0)
        pos_ge_v = _splat(jnp.where(pos_key >= kth, 1.0, 0.0), jnp.float32)
        total_v = (_splat(s_all, jnp.float32) - pos_ge_v * e_pos_v
                   + (float(_K) - _splat(cnt_ge, jnp.float32))
                   * e_thr_v + e_pos_v)

        acc = jnp.where(iota == _splat(j), total_v, acc)
        acc = jnp.where(iota == _splat(_RPW + j), mx_v, acc)
        acc = jnp.where(iota == _splat(2 * _RPW + j), pos_v, acc)

    out_v[...] = acc
    pltpu.sync_copy(out_v, out_hbm.at[wid])


def _sc_stage(inputs, targets):
    mesh = plsc.VectorSubcoreMesh(core_axis_name="c", subcore_axis_name="s",
                                  num_cores=_NC)
    return pl.kernel(
        _sc_body,
        out_type=jax.ShapeDtypeStruct((_NW, _L), jnp.float32),
        mesh=mesh,
        scratch_types=[
            pltpu.VMEM((_N,), jnp.float32),
            pltpu.VMEM((_N,), jnp.int32),
            pltpu.VMEM((_M,), jnp.int32),
            pltpu.VMEM((_L,), jnp.float32),
            pltpu.VMEM((2 * _L,), jnp.float32),
            pltpu.VMEM((2 * _L,), jnp.int32),
        ],
    )(inputs, targets)


def _tc_body(x_ref, t_ref, o_ref, keys_ref):
    x = x_ref[...]
    tgt = t_ref[...]
    col = lax.broadcasted_iota(jnp.int32, (_TB, _N), 1)
    ispos = col == tgt
    keys_ref[...] = jnp.where(
        ispos, _INT_MIN, _to_key(lax.bitcast_convert_type(x, jnp.int32)))
    mx = jnp.max(x, axis=1, keepdims=True)
    pos = jnp.sum(jnp.where(ispos, x, 0.0), axis=1, keepdims=True)

    def search_body(i, p):
        cand = jnp.where(i == 0, jnp.zeros_like(p),
                         p | (jnp.int32(1) << (31 - i)))
        cnt = jnp.sum((keys_ref[...] >= cand).astype(jnp.float32),
                      axis=1, keepdims=True)
        return jnp.where(cnt >= float(_K), cand, p)

    kth = lax.fori_loop(
        0, 32, search_body, jnp.full((_TB, 1), _INT_MIN, jnp.int32))

    keys = keys_ref[...]
    gt = keys > kth
    e = jnp.exp((x - mx) * 10.0)
    s_gt = jnp.sum(jnp.where(gt, e, 0.0), axis=1, keepdims=True)
    cnt_gt = jnp.sum(gt.astype(jnp.float32), axis=1, keepdims=True)
    thr = lax.bitcast_convert_type(_to_key(kth), jnp.float32)
    e_thr = jnp.exp((thr - mx) * 10.0)
    e_pos = jnp.exp((pos - mx) * 10.0)
    total = s_gt + (float(_K) - cnt_gt) * e_thr + e_pos
    o_ref[...] = jnp.log(total) + 10.0 * (mx - pos)


def _tc_stage(inputs, targets_2d):
    nblk = (_M - _RS) // _TB
    return pl.pallas_call(
        _tc_body,
        grid=(nblk,),
        in_specs=[
            pl.BlockSpec((_TB, _N), lambda i: (i + _RS // _TB, 0)),
            pl.BlockSpec((_TB, 1), lambda i: (i + _RS // _TB, 0)),
        ],
        out_specs=pl.BlockSpec((_TB, 1), lambda i: (i, 0)),
        out_shape=jax.ShapeDtypeStruct((_M - _RS, 1), jnp.float32),
        scratch_shapes=[pltpu.VMEM((_TB, _N), jnp.int32)],
    )(inputs, targets_2d)


def _finish_body(sc_ref, tc_ref, o_ref):
    sc = sc_ref[...]
    s = sc[:, 0:_RPW]
    mx = sc[:, _RPW:2 * _RPW]
    pos = sc[:, 2 * _RPW:3 * _RPW]
    sc_loss = jnp.log(s) + 10.0 * (mx - pos)
    o_ref[0] = (jnp.sum(sc_loss) + jnp.sum(tc_ref[...])) * (1.0 / _M)


def _finish(sc_out, tc_out):
    return pl.pallas_call(
        _finish_body,
        out_shape=jax.ShapeDtypeStruct((1,), jnp.float32),
        out_specs=pl.BlockSpec(memory_space=pltpu.SMEM),
    )(sc_out, tc_out)


@jax.jit
def _run(inputs, targets):
    t32 = targets.astype(jnp.int32)
    sc_out = _sc_stage(inputs, t32)
    tc_out = _tc_stage(inputs, t32.reshape(_M, 1))
    return _finish(sc_out, tc_out)[0]


def kernel(inputs, targets):
    return _run(inputs, targets)
